# baseline (speedup 1.0000x reference)
import jax
import jax.numpy as jnp
from jax import lax
from jax.experimental import pallas as pl
from jax.experimental.pallas import tpu as pltpu
from jax.experimental.pallas import tpu_sc as plsc

BATCH = 4
MAX_LEN = 2048
EMBED = 1024
NC, NS, L = 2, 16, 16
NW = NC * NS
T_PER_W = MAX_LEN // NW
CT = 8
NCHUNK = T_PER_W // CT
VREGS_PER_ROW = EMBED // L


def _body(seq_hbm, tok_hbm, pos_hbm, out_hbm,
          idx_v, rows0, rows1, pos0, pos1, gsem0, gsem1, ssem0, ssem1):
    wid = lax.axis_index("s") * NC + lax.axis_index("c")
    tw0 = wid * T_PER_W
    for b in range(BATCH):
        pltpu.sync_copy(seq_hbm.at[b, pl.ds(tw0, T_PER_W)], idx_v.at[b])

    rows = [rows0, rows1]
    pos = [pos0, pos1]
    gsem = [gsem0, gsem1]
    ssem = [ssem0, ssem1]

    def start_unit(c):
        buf = c % 2
        t0 = tw0 + c * CT
        descs = [pltpu.async_copy(pos_hbm.at[pl.ds(t0, CT)], pos[buf], gsem[buf])]
        for b in range(BATCH):
            descs.append(pltpu.async_copy(
                tok_hbm.at[idx_v.at[b, pl.ds(c * CT, CT)]],
                rows[buf].at[b], gsem[buf]))
        return descs

    pend_g = {0: start_unit(0)}
    pend_s = {}
    for c in range(NCHUNK):
        buf = c % 2
        nxt = c + 1
        if nxt < NCHUNK:
            for d in pend_s.pop(nxt % 2, ()):
                d.wait()
            pend_g[nxt] = start_unit(nxt)
        for d in pend_g.pop(c):
            d.wait()

        @plsc.parallel_loop(0, VREGS_PER_ROW, unroll=4)
        def add_j(j, _buf=buf):
            sl = pl.ds(j * L, L)
            for r in range(CT):
                p = pos[_buf][r, sl]
                for b in range(BATCH):
                    rows[_buf][b, r, sl] = rows[_buf][b, r, sl] + p

        t0 = tw0 + c * CT
        pend_s[buf] = [
            pltpu.async_copy(rows[buf].at[b], out_hbm.at[b, pl.ds(t0, CT)],
                             ssem[buf])
            for b in range(BATCH)
        ]
    for descs in pend_s.values():
        for d in descs:
            d.wait()


def kernel(sequence, token_weight, position_weight):
    mesh = plsc.VectorSubcoreMesh(core_axis_name="c", subcore_axis_name="s")
    f = pl.kernel(
        _body,
        out_type=jax.ShapeDtypeStruct((BATCH, MAX_LEN, EMBED), jnp.float32),
        mesh=mesh,
        scratch_types=[
            pltpu.VMEM((BATCH, T_PER_W), jnp.int32),
            pltpu.VMEM((BATCH, CT, EMBED), jnp.float32),
            pltpu.VMEM((BATCH, CT, EMBED), jnp.float32),
            pltpu.VMEM((CT, EMBED), jnp.float32),
            pltpu.VMEM((CT, EMBED), jnp.float32),
            pltpu.SemaphoreType.DMA,
            pltpu.SemaphoreType.DMA,
            pltpu.SemaphoreType.DMA,
            pltpu.SemaphoreType.DMA,
        ],
    )
    return f(sequence, token_weight, position_weight)


# speedup vs baseline: 1.1325x; 1.1325x over previous
import jax
import jax.numpy as jnp
from jax import lax
from jax.experimental import pallas as pl
from jax.experimental.pallas import tpu as pltpu
from jax.experimental.pallas import tpu_sc as plsc

BATCH = 4
MAX_LEN = 2048
EMBED = 1024
NC, NS, L = 2, 16, 16
NW = NC * NS
T_PER_W = MAX_LEN // NW
CT = 8
NCHUNK = T_PER_W // CT
NBUF = 3
VREGS_PER_ROW = EMBED // L


def _body(seq_hbm, tok_hbm, pos_hbm, out_hbm, idx_v,
          rows0, rows1, rows2, pos0, pos1, pos2,
          gsem0, gsem1, gsem2, ssem0, ssem1, ssem2):
    wid = lax.axis_index("s") * NC + lax.axis_index("c")
    tw0 = wid * T_PER_W
    for b in range(BATCH):
        pltpu.sync_copy(seq_hbm.at[b, pl.ds(tw0, T_PER_W)], idx_v.at[b])

    rows = [rows0, rows1, rows2]
    pos = [pos0, pos1, pos2]
    gsem = [gsem0, gsem1, gsem2]
    ssem = [ssem0, ssem1, ssem2]

    def start_unit(c):
        buf = c % NBUF
        t0 = tw0 + c * CT
        descs = [pltpu.async_copy(pos_hbm.at[pl.ds(t0, CT)], pos[buf], gsem[buf])]
        for b in range(BATCH):
            descs.append(pltpu.async_copy(
                tok_hbm.at[idx_v.at[b, pl.ds(c * CT, CT)]],
                rows[buf].at[b], gsem[buf]))
        return descs

    pend_g = {c: start_unit(c) for c in range(NBUF - 1)}
    pend_s = {}
    for c in range(NCHUNK):
        buf = c % NBUF
        nxt = c + NBUF - 1
        if nxt < NCHUNK:
            for d in pend_s.pop(nxt % NBUF, ()):
                d.wait()
            pend_g[nxt] = start_unit(nxt)
        for d in pend_g.pop(c):
            d.wait()

        def add_j(j, carry, _buf=buf):
            sl = pl.ds(j * L, L)
            for r in range(CT):
                p = pos[_buf][r, sl]
                for b in range(BATCH):
                    rows[_buf][b, r, sl] = rows[_buf][b, r, sl] + p
            return carry

        lax.fori_loop(0, VREGS_PER_ROW, add_j, 0)

        t0 = tw0 + c * CT
        pend_s[buf] = [
            pltpu.async_copy(rows[buf].at[b], out_hbm.at[b, pl.ds(t0, CT)],
                             ssem[buf])
            for b in range(BATCH)
        ]
    for descs in pend_s.values():
        for d in descs:
            d.wait()


def kernel(sequence, token_weight, position_weight):
    mesh = plsc.VectorSubcoreMesh(core_axis_name="c", subcore_axis_name="s")
    f = pl.kernel(
        _body,
        out_type=jax.ShapeDtypeStruct((BATCH, MAX_LEN, EMBED), jnp.float32),
        mesh=mesh,
        scratch_types=[
            pltpu.VMEM((BATCH, T_PER_W), jnp.int32),
            pltpu.VMEM((BATCH, CT, EMBED), jnp.float32),
            pltpu.VMEM((BATCH, CT, EMBED), jnp.float32),
            pltpu.VMEM((BATCH, CT, EMBED), jnp.float32),
            pltpu.VMEM((CT, EMBED), jnp.float32),
            pltpu.VMEM((CT, EMBED), jnp.float32),
            pltpu.VMEM((CT, EMBED), jnp.float32),
            pltpu.SemaphoreType.DMA,
            pltpu.SemaphoreType.DMA,
            pltpu.SemaphoreType.DMA,
            pltpu.SemaphoreType.DMA,
            pltpu.SemaphoreType.DMA,
            pltpu.SemaphoreType.DMA,
        ],
    )
    return f(sequence, token_weight, position_weight)


# speedup vs baseline: 1.1368x; 1.0038x over previous
import jax
import jax.numpy as jnp
from jax import lax
from jax.experimental import pallas as pl
from jax.experimental.pallas import tpu as pltpu
from jax.experimental.pallas import tpu_sc as plsc

BATCH = 4
MAX_LEN = 2048
EMBED = 1024
NC, NS, L = 2, 16, 16
NW = NC * NS
T_PER_W = MAX_LEN // NW
CT = 16
NQ = T_PER_W // CT
NPAIR = 2
VREGS_PER_ROW = EMBED // L


def _body(seq_hbm, tok_hbm, pos_hbm, out_hbm, idx_v,
          rows0, rows1, pos0, pos1,
          gsem0, gsem1, psem0, psem1, ssem0, ssem1):
    wid = lax.axis_index("s") * NC + lax.axis_index("c")
    tw0 = wid * T_PER_W
    for b in range(BATCH):
        pltpu.sync_copy(seq_hbm.at[b, pl.ds(tw0, T_PER_W)], idx_v.at[b])

    rows = [rows0, rows1]
    pos = [pos0, pos1]
    gsem = [gsem0, gsem1]
    psem = [psem0, psem1]
    ssem = [ssem0, ssem1]

    def start_pos(q):
        pb = q % 2
        return [pltpu.async_copy(pos_hbm.at[pl.ds(tw0 + q * CT, CT)],
                                 pos[pb], psem[pb])]

    def start_unit(u):
        q, pr = divmod(u, NPAIR)
        rb = u % 2
        return [
            pltpu.async_copy(
                tok_hbm.at[idx_v.at[2 * pr + i, pl.ds(q * CT, CT)]],
                rows[rb].at[i], gsem[rb])
            for i in range(2)
        ]

    NU = NQ * NPAIR
    pend_pos = {0: start_pos(0), 1: start_pos(1)}
    pend_g = {0: start_unit(0), 1: start_unit(1)}
    pend_s = {}
    pos_waited = [False, False]
    for u in range(NU):
        q, pr = divmod(u, NPAIR)
        rb = u % 2
        pb = q % 2
        nxt = u + 2
        if nxt < NU:
            for d in pend_s.pop(rb, ()):
                d.wait()
            pend_g[nxt] = start_unit(nxt)
        for d in pend_g.pop(u):
            d.wait()
        if not pos_waited[pb]:
            for d in pend_pos.pop(q):
                d.wait()
            pos_waited[pb] = True

        def add_j(j, carry, _rb=rb, _pb=pb):
            sl = pl.ds(j * L, L)
            for r in range(CT):
                p = pos[_pb][r, sl]
                for i in range(2):
                    rows[_rb][i, r, sl] = rows[_rb][i, r, sl] + p
            return carry

        lax.fori_loop(0, VREGS_PER_ROW, add_j, 0)

        if pr == NPAIR - 1 and q + 2 < NQ:
            pend_pos[q + 2] = start_pos(q + 2)
            pos_waited[pb] = False

        t0 = tw0 + q * CT
        pend_s[rb] = [
            pltpu.async_copy(rows[rb].at[i], out_hbm.at[2 * pr + i, pl.ds(t0, CT)],
                             ssem[rb])
            for i in range(2)
        ]
    for descs in pend_s.values():
        for d in descs:
            d.wait()


def kernel(sequence, token_weight, position_weight):
    mesh = plsc.VectorSubcoreMesh(core_axis_name="c", subcore_axis_name="s")
    f = pl.kernel(
        _body,
        out_type=jax.ShapeDtypeStruct((BATCH, MAX_LEN, EMBED), jnp.float32),
        mesh=mesh,
        scratch_types=[
            pltpu.VMEM((BATCH, T_PER_W), jnp.int32),
            pltpu.VMEM((NPAIR, CT, EMBED), jnp.float32),
            pltpu.VMEM((NPAIR, CT, EMBED), jnp.float32),
            pltpu.VMEM((CT, EMBED), jnp.float32),
            pltpu.VMEM((CT, EMBED), jnp.float32),
            pltpu.SemaphoreType.DMA,
            pltpu.SemaphoreType.DMA,
            pltpu.SemaphoreType.DMA,
            pltpu.SemaphoreType.DMA,
            pltpu.SemaphoreType.DMA,
            pltpu.SemaphoreType.DMA,
        ],
    )
    return f(sequence, token_weight, position_weight)
